# baseline (speedup 1.0000x reference)
import functools

import jax
import jax.numpy as jnp
from jax import lax
from jax.experimental import pallas as pl
from jax.experimental.pallas import tpu as pltpu
from jax.experimental.pallas import tpu_sc as plsc

N_G = 1024
H_IMG = 128
W_IMG = 128
FX = 110.9
FY = 110.9
CX = 64.0
CY = 64.0
NEAR = 0.01
FAR = 100.0
MAX_RADIUS = 32.0


def _b16(x):
    return x.astype(jnp.bfloat16).astype(jnp.float32)


def _project_kernel(view_ref, px_ref, py_ref, pz_ref, sx_ref, sy_ref, sz_ref,
                    qw_ref, qx_ref, qy_ref, qz_ref, op_ref, out_ref):
    v = view_ref
    vb = [[_b16(v[i, j]) for j in range(4)] for i in range(4)]
    px = _b16(px_ref[...])
    py = _b16(py_ref[...])
    pz = _b16(pz_ref[...])
    pcx = vb[0][0] * px + vb[0][1] * py + vb[0][2] * pz + vb[0][3]
    pcy = vb[1][0] * px + vb[1][1] * py + vb[1][2] * pz + vb[1][3]
    pcz = vb[2][0] * px + vb[2][1] * py + vb[2][2] * pz + vb[2][3]
    depth = -pcz

    qw = qw_ref[...]
    qx = qx_ref[...]
    qy = qy_ref[...]
    qz = qz_ref[...]
    qn = jnp.sqrt(qw * qw + qx * qx + qy * qy + qz * qz) + 1e-12
    w = qw / qn
    x = qx / qn
    y = qy / qn
    z = qz / qn
    r = [[1 - 2 * y * y - 2 * z * z, 2 * x * y - 2 * w * z, 2 * x * z + 2 * w * y],
         [2 * x * y + 2 * w * z, 1 - 2 * x * x - 2 * z * z, 2 * y * z - 2 * w * x],
         [2 * x * z - 2 * w * y, 2 * y * z + 2 * w * x, 1 - 2 * x * x - 2 * y * y]]
    s = [_b16(sx_ref[...]), _b16(sy_ref[...]), _b16(sz_ref[...])]
    rc = [[vb[i][0] * _b16(r[0][j]) + vb[i][1] * _b16(r[1][j])
           + vb[i][2] * _b16(r[2][j]) for j in range(3)] for i in range(3)]
    rs = [[_b16(rc[i][j]) * s[j] for j in range(3)] for i in range(3)]
    rsb = [[_b16(rs[i][j]) for j in range(3)] for i in range(3)]
    cov = [[rsb[i][0] * rsb[j][0] + rsb[i][1] * rsb[j][1] + rsb[i][2] * rsb[j][2]
            for j in range(3)] for i in range(3)]

    zsafe = jnp.maximum(jnp.abs(pcz), 0.01) * jnp.sign(pcz + 1e-8)
    z2 = zsafe * zsafe
    j00 = FX / -zsafe
    j02 = FX * pcx / z2
    j11 = FY / zsafe
    j12 = FY * pcy / z2
    j00b = _b16(j00)
    j02b = _b16(j02)
    j11b = _b16(j11)
    j12b = _b16(j12)
    covb = [[_b16(cov[i][j]) for j in range(3)] for i in range(3)]
    t00 = j00b * covb[0][0] + j02b * covb[2][0]
    t01 = j00b * covb[0][1] + j02b * covb[2][1]
    t02 = j00b * covb[0][2] + j02b * covb[2][2]
    t10 = j11b * covb[1][0] + j12b * covb[2][0]
    t11 = j11b * covb[1][1] + j12b * covb[2][1]
    t12 = j11b * covb[1][2] + j12b * covb[2][2]
    a = _b16(t00) * j00b + _b16(t02) * j02b
    b = _b16(t01) * j11b + _b16(t02) * j12b
    c = _b16(t10) * j00b + _b16(t12) * j02b
    d = _b16(t11) * j11b + _b16(t12) * j12b

    u = FX * pcx / -zsafe + CX
    vv = FY * -pcy / -zsafe + CY
    trace = a + d
    det = jnp.maximum(a * d - b * c, 1e-6)
    disc = jnp.maximum(trace * trace - 4.0 * det, 0.0)
    max_eig = (trace + jnp.sqrt(disc)) / 2.0
    radii = jnp.minimum(3.0 * jnp.sqrt(jnp.maximum(max_eig, 1e-6)), MAX_RADIUS)

    vis = ((depth > NEAR) & (depth < FAR)
           & (u + radii > 0) & (u - radii < W_IMG)
           & (vv + radii > 0) & (vv - radii < H_IMG))

    ar = a + 0.3
    dr = d + 0.3
    br = b
    det_r = jnp.maximum(ar * dr - br * br, 1e-6)
    inv_a = dr / det_r
    inv_d = ar / det_r
    inv_b = -br / det_r
    oe = op_ref[...] * vis.astype(jnp.float32)

    out_ref[0] = depth
    out_ref[1] = u
    out_ref[2] = vv
    out_ref[3] = inv_a
    out_ref[4] = inv_b
    out_ref[5] = inv_d
    out_ref[6] = oe
    out_ref[7] = radii


def _rank_kernel(dcol_ref, drow_ref, icol_ref, irow_ref, out_ref):
    dcol = dcol_ref[...]
    drow = drow_ref[...]
    icol = icol_ref[...]
    irow = irow_ref[...]
    lt = jnp.where((dcol < drow) | ((dcol == drow) & (icol < irow)), 1.0, 0.0)
    out_ref[...] = jnp.sum(lt, axis=0, keepdims=True)


_SC_TILES = 32
_ROWS_PER_TILE = N_G // _SC_TILES


_MCOLS = 128


def _sc_sort_scatter(m_hbm, rank_hbm, out_hbm, idx_v, rows_v, sem):
    wid = lax.axis_index("s") * 2 + lax.axis_index("c")
    base = wid * _ROWS_PER_TILE
    pltpu.sync_copy(rank_hbm.at[pl.ds(base, _ROWS_PER_TILE)], idx_v)
    pltpu.sync_copy(m_hbm.at[pl.ds(base, _ROWS_PER_TILE)], rows_v)
    pltpu.async_copy(rows_v, out_hbm.at[idx_v], sem).wait()


def _sorted_params(depth, m):
    f32 = jnp.float32
    dcol = depth.reshape(N_G, 1)
    drow = depth.reshape(1, N_G)
    idx = jnp.arange(N_G, dtype=f32)
    icol = idx.reshape(N_G, 1)
    irow = idx.reshape(1, N_G)
    rank = pl.pallas_call(
        _rank_kernel,
        out_shape=jax.ShapeDtypeStruct((1, N_G), f32),
    )(dcol, drow, icol, irow)
    rank_i = rank.reshape(N_G).astype(jnp.int32)

    mesh = plsc.VectorSubcoreMesh(core_axis_name="c", subcore_axis_name="s")
    scatter = functools.partial(
        pl.kernel, mesh=mesh,
        out_type=jax.ShapeDtypeStruct((N_G, _MCOLS), f32),
        scratch_types=[
            pltpu.VMEM((_ROWS_PER_TILE,), jnp.int32),
            pltpu.VMEM((_ROWS_PER_TILE, _MCOLS), f32),
            pltpu.SemaphoreType.DMA,
        ],
    )(_sc_sort_scatter)
    return scatter(m, rank_i)[:, :16]


_WROWS = 80


def _raster_kernel(ms_ref, out_ref, t_ref):
    t_ref[...] = jnp.ones((H_IMG, W_IMG), jnp.float32)
    out_ref[...] = jnp.zeros((3, H_IMG, W_IMG), jnp.float32)

    def body(g, carry):
        u = ms_ref[g, 0]
        v = ms_ref[g, 1]
        inv_a = ms_ref[g, 2]
        inv_b = ms_ref[g, 3]
        inv_d = ms_ref[g, 4]
        oe = ms_ref[g, 5]
        rad = ms_ref[g, 6]
        c0b = _b16(ms_ref[g, 7])
        c1b = _b16(ms_ref[g, 8])
        c2b = _b16(ms_ref[g, 9])
        r2 = rad * rad

        lo = jnp.maximum(jnp.floor(v - rad).astype(jnp.int32) - 1, 0)
        lo = (lo // 8) * 8
        start = pl.multiple_of(
            jnp.clip(lo, 0, H_IMG - _WROWS), 8)
        ys = (jax.lax.broadcasted_iota(jnp.int32, (_WROWS, W_IMG), 0)
              + start).astype(jnp.float32) + 0.5
        xs = jax.lax.broadcasted_iota(
            jnp.int32, (_WROWS, W_IMG), 1).astype(jnp.float32) + 0.5
        dx = xs - u
        dy = ys - v
        dx2 = dx * dx
        dy2 = dy * dy
        power = -0.5 * (inv_a * dx2 + inv_d * dy2) - inv_b * (dx * dy)
        power = jnp.minimum(power, 0.0)
        gauss = jnp.exp(power)
        alpha = jnp.where(dx2 + dy2 <= r2, oe * gauss, 0.0)
        alpha = jnp.clip(alpha, 0.0, 0.99)
        tcur = t_ref[pl.ds(start, _WROWS), :]
        wgt = _b16(tcur * alpha)
        out_ref[0, pl.ds(start, _WROWS), :] += wgt * c0b
        out_ref[1, pl.ds(start, _WROWS), :] += wgt * c1b
        out_ref[2, pl.ds(start, _WROWS), :] += wgt * c2b
        t_ref[pl.ds(start, _WROWS), :] = tcur * (1.0 - alpha)
        return carry

    jax.lax.fori_loop(0, N_G, body, 0)


def kernel(positions, scales, rotations, colors, opacities, view_matrix):
    f32 = jnp.float32
    px = positions[:, 0].reshape(8, 128)
    py = positions[:, 1].reshape(8, 128)
    pz = positions[:, 2].reshape(8, 128)
    sx = scales[:, 0].reshape(8, 128)
    sy = scales[:, 1].reshape(8, 128)
    sz = scales[:, 2].reshape(8, 128)
    qw = rotations[:, 0].reshape(8, 128)
    qx = rotations[:, 1].reshape(8, 128)
    qy = rotations[:, 2].reshape(8, 128)
    qz = rotations[:, 3].reshape(8, 128)
    op = opacities.reshape(8, 128)

    proj = pl.pallas_call(
        _project_kernel,
        out_shape=jax.ShapeDtypeStruct((8, 8, 128), f32),
        in_specs=[pl.BlockSpec(memory_space=pltpu.SMEM)]
                 + [pl.BlockSpec(memory_space=pltpu.VMEM)] * 11,
    )(view_matrix, px, py, pz, sx, sy, sz, qw, qx, qy, qz, op)

    flat = proj.reshape(8, N_G)
    depth = flat[0]
    m = jnp.concatenate(
        [flat[1:8].T, colors, jnp.zeros((N_G, _MCOLS - 10), f32)], axis=1)
    ms = _sorted_params(depth, m)

    img = pl.pallas_call(
        _raster_kernel,
        out_shape=jax.ShapeDtypeStruct((3, H_IMG, W_IMG), f32),
        in_specs=[pl.BlockSpec(memory_space=pltpu.SMEM)],
        scratch_shapes=[pltpu.VMEM((H_IMG, W_IMG), f32)],
    )(ms)
    return jnp.transpose(img, (1, 2, 0))


# speedup vs baseline: 1.1320x; 1.1320x over previous
import functools

import jax
import jax.numpy as jnp
from jax import lax
from jax.experimental import pallas as pl
from jax.experimental.pallas import tpu as pltpu
from jax.experimental.pallas import tpu_sc as plsc

N_G = 1024
H_IMG = 128
W_IMG = 128
FX = 110.9
FY = 110.9
CX = 64.0
CY = 64.0
NEAR = 0.01
FAR = 100.0
MAX_RADIUS = 32.0


def _b16(x):
    return x.astype(jnp.bfloat16).astype(jnp.float32)


def _project_kernel(view_ref, px_ref, py_ref, pz_ref, sx_ref, sy_ref, sz_ref,
                    qw_ref, qx_ref, qy_ref, qz_ref, op_ref, out_ref):
    v = view_ref
    vb = [[_b16(v[i, j]) for j in range(4)] for i in range(4)]
    px = _b16(px_ref[...])
    py = _b16(py_ref[...])
    pz = _b16(pz_ref[...])
    pcx = vb[0][0] * px + vb[0][1] * py + vb[0][2] * pz + vb[0][3]
    pcy = vb[1][0] * px + vb[1][1] * py + vb[1][2] * pz + vb[1][3]
    pcz = vb[2][0] * px + vb[2][1] * py + vb[2][2] * pz + vb[2][3]
    depth = -pcz

    qw = qw_ref[...]
    qx = qx_ref[...]
    qy = qy_ref[...]
    qz = qz_ref[...]
    qn = jnp.sqrt(qw * qw + qx * qx + qy * qy + qz * qz) + 1e-12
    w = qw / qn
    x = qx / qn
    y = qy / qn
    z = qz / qn
    r = [[1 - 2 * y * y - 2 * z * z, 2 * x * y - 2 * w * z, 2 * x * z + 2 * w * y],
         [2 * x * y + 2 * w * z, 1 - 2 * x * x - 2 * z * z, 2 * y * z - 2 * w * x],
         [2 * x * z - 2 * w * y, 2 * y * z + 2 * w * x, 1 - 2 * x * x - 2 * y * y]]
    s = [_b16(sx_ref[...]), _b16(sy_ref[...]), _b16(sz_ref[...])]
    rc = [[vb[i][0] * _b16(r[0][j]) + vb[i][1] * _b16(r[1][j])
           + vb[i][2] * _b16(r[2][j]) for j in range(3)] for i in range(3)]
    rs = [[_b16(rc[i][j]) * s[j] for j in range(3)] for i in range(3)]
    rsb = [[_b16(rs[i][j]) for j in range(3)] for i in range(3)]
    cov = [[rsb[i][0] * rsb[j][0] + rsb[i][1] * rsb[j][1] + rsb[i][2] * rsb[j][2]
            for j in range(3)] for i in range(3)]

    zsafe = jnp.maximum(jnp.abs(pcz), 0.01) * jnp.sign(pcz + 1e-8)
    z2 = zsafe * zsafe
    j00 = FX / -zsafe
    j02 = FX * pcx / z2
    j11 = FY / zsafe
    j12 = FY * pcy / z2
    j00b = _b16(j00)
    j02b = _b16(j02)
    j11b = _b16(j11)
    j12b = _b16(j12)
    covb = [[_b16(cov[i][j]) for j in range(3)] for i in range(3)]
    t00 = j00b * covb[0][0] + j02b * covb[2][0]
    t01 = j00b * covb[0][1] + j02b * covb[2][1]
    t02 = j00b * covb[0][2] + j02b * covb[2][2]
    t10 = j11b * covb[1][0] + j12b * covb[2][0]
    t11 = j11b * covb[1][1] + j12b * covb[2][1]
    t12 = j11b * covb[1][2] + j12b * covb[2][2]
    a = _b16(t00) * j00b + _b16(t02) * j02b
    b = _b16(t01) * j11b + _b16(t02) * j12b
    c = _b16(t10) * j00b + _b16(t12) * j02b
    d = _b16(t11) * j11b + _b16(t12) * j12b

    u = FX * pcx / -zsafe + CX
    vv = FY * -pcy / -zsafe + CY
    trace = a + d
    det = jnp.maximum(a * d - b * c, 1e-6)
    disc = jnp.maximum(trace * trace - 4.0 * det, 0.0)
    max_eig = (trace + jnp.sqrt(disc)) / 2.0
    radii = jnp.minimum(3.0 * jnp.sqrt(jnp.maximum(max_eig, 1e-6)), MAX_RADIUS)

    vis = ((depth > NEAR) & (depth < FAR)
           & (u + radii > 0) & (u - radii < W_IMG)
           & (vv + radii > 0) & (vv - radii < H_IMG))

    ar = a + 0.3
    dr = d + 0.3
    br = b
    det_r = jnp.maximum(ar * dr - br * br, 1e-6)
    inv_a = dr / det_r
    inv_d = ar / det_r
    inv_b = -br / det_r
    oe = op_ref[...] * vis.astype(jnp.float32)

    out_ref[0] = depth
    out_ref[1] = u
    out_ref[2] = vv
    out_ref[3] = inv_a
    out_ref[4] = inv_b
    out_ref[5] = inv_d
    out_ref[6] = oe
    out_ref[7] = radii


def _rank_kernel(dcol_ref, drow_ref, icol_ref, irow_ref, out_ref):
    dcol = dcol_ref[...]
    drow = drow_ref[...]
    icol = icol_ref[...]
    irow = irow_ref[...]
    lt = jnp.where((dcol < drow) | ((dcol == drow) & (icol < irow)), 1.0, 0.0)
    out_ref[...] = jnp.sum(lt, axis=0, keepdims=True)


_SC_TILES = 32
_ROWS_PER_TILE = N_G // _SC_TILES


_MCOLS = 128


def _sc_sort_scatter(m_hbm, rank_hbm, out_hbm, idx_v, rows_v, sem):
    wid = lax.axis_index("s") * 2 + lax.axis_index("c")
    base = wid * _ROWS_PER_TILE
    pltpu.sync_copy(rank_hbm.at[pl.ds(base, _ROWS_PER_TILE)], idx_v)
    pltpu.sync_copy(m_hbm.at[pl.ds(base, _ROWS_PER_TILE)], rows_v)
    pltpu.async_copy(rows_v, out_hbm.at[idx_v], sem).wait()


def _sorted_params(depth, m):
    f32 = jnp.float32
    dcol = depth.reshape(N_G, 1)
    drow = depth.reshape(1, N_G)
    idx = jnp.arange(N_G, dtype=f32)
    icol = idx.reshape(N_G, 1)
    irow = idx.reshape(1, N_G)
    rank = pl.pallas_call(
        _rank_kernel,
        out_shape=jax.ShapeDtypeStruct((1, N_G), f32),
    )(dcol, drow, icol, irow)
    rank_i = rank.reshape(N_G).astype(jnp.int32)

    mesh = plsc.VectorSubcoreMesh(core_axis_name="c", subcore_axis_name="s")
    scatter = functools.partial(
        pl.kernel, mesh=mesh,
        out_type=jax.ShapeDtypeStruct((N_G, _MCOLS), f32),
        scratch_types=[
            pltpu.VMEM((_ROWS_PER_TILE,), jnp.int32),
            pltpu.VMEM((_ROWS_PER_TILE, _MCOLS), f32),
            pltpu.SemaphoreType.DMA,
        ],
    )(_sc_sort_scatter)
    return scatter(m, rank_i)[:, :16]


_WROWS = 80


def _raster_kernel(ms_ref, out_ref, t_ref):
    t_ref[...] = jnp.ones((H_IMG, W_IMG), jnp.float32)
    out_ref[...] = jnp.zeros((3, H_IMG, W_IMG), jnp.float32)

    xs = jax.lax.broadcasted_iota(
        jnp.int32, (_WROWS, W_IMG), 1).astype(jnp.float32) + 0.5

    def alpha_for(g):
        u = ms_ref[g, 0]
        v = ms_ref[g, 1]
        inv_a = ms_ref[g, 2]
        inv_b = ms_ref[g, 3]
        inv_d = ms_ref[g, 4]
        oe = ms_ref[g, 5]
        rad = ms_ref[g, 6]
        r2 = rad * rad
        lo = jnp.maximum(jnp.floor(v - rad).astype(jnp.int32) - 1, 0)
        lo = (lo // 8) * 8
        start = pl.multiple_of(jnp.clip(lo, 0, H_IMG - _WROWS), 8)
        ys = (jax.lax.broadcasted_iota(jnp.int32, (_WROWS, W_IMG), 0)
              + start).astype(jnp.float32) + 0.5
        dx = xs - u
        dy = ys - v
        dx2 = dx * dx
        dy2 = dy * dy
        power = -0.5 * (inv_a * dx2 + inv_d * dy2) - inv_b * (dx * dy)
        power = jnp.minimum(power, 0.0)
        gauss = jnp.exp(power)
        alpha = jnp.where(dx2 + dy2 <= r2, oe * gauss, 0.0)
        return start, jnp.minimum(alpha, 0.99)

    def apply(g, start, alpha):
        c0b = _b16(ms_ref[g, 7])
        c1b = _b16(ms_ref[g, 8])
        c2b = _b16(ms_ref[g, 9])
        tcur = t_ref[pl.ds(start, _WROWS), :]
        wgt = _b16(tcur * alpha)
        out_ref[0, pl.ds(start, _WROWS), :] += wgt * c0b
        out_ref[1, pl.ds(start, _WROWS), :] += wgt * c1b
        out_ref[2, pl.ds(start, _WROWS), :] += wgt * c2b
        t_ref[pl.ds(start, _WROWS), :] = tcur * (1.0 - alpha)

    def body(gp, carry):
        g0 = 2 * gp
        g1 = g0 + 1
        s0, a0 = alpha_for(g0)
        s1, a1 = alpha_for(g1)
        apply(g0, s0, a0)
        apply(g1, s1, a1)
        return carry

    jax.lax.fori_loop(0, N_G // 2, body, 0)


def kernel(positions, scales, rotations, colors, opacities, view_matrix):
    f32 = jnp.float32
    px = positions[:, 0].reshape(8, 128)
    py = positions[:, 1].reshape(8, 128)
    pz = positions[:, 2].reshape(8, 128)
    sx = scales[:, 0].reshape(8, 128)
    sy = scales[:, 1].reshape(8, 128)
    sz = scales[:, 2].reshape(8, 128)
    qw = rotations[:, 0].reshape(8, 128)
    qx = rotations[:, 1].reshape(8, 128)
    qy = rotations[:, 2].reshape(8, 128)
    qz = rotations[:, 3].reshape(8, 128)
    op = opacities.reshape(8, 128)

    proj = pl.pallas_call(
        _project_kernel,
        out_shape=jax.ShapeDtypeStruct((8, 8, 128), f32),
        in_specs=[pl.BlockSpec(memory_space=pltpu.SMEM)]
                 + [pl.BlockSpec(memory_space=pltpu.VMEM)] * 11,
    )(view_matrix, px, py, pz, sx, sy, sz, qw, qx, qy, qz, op)

    flat = proj.reshape(8, N_G)
    depth = flat[0]
    m = jnp.concatenate(
        [flat[1:8].T, colors, jnp.zeros((N_G, _MCOLS - 10), f32)], axis=1)
    ms = _sorted_params(depth, m)

    img = pl.pallas_call(
        _raster_kernel,
        out_shape=jax.ShapeDtypeStruct((3, H_IMG, W_IMG), f32),
        in_specs=[pl.BlockSpec(memory_space=pltpu.SMEM)],
        scratch_shapes=[pltpu.VMEM((H_IMG, W_IMG), f32)],
    )(ms)
    return jnp.transpose(img, (1, 2, 0))


# speedup vs baseline: 1.1748x; 1.0378x over previous
import functools

import jax
import jax.numpy as jnp
from jax import lax
from jax.experimental import pallas as pl
from jax.experimental.pallas import tpu as pltpu
from jax.experimental.pallas import tpu_sc as plsc

N_G = 1024
H_IMG = 128
W_IMG = 128
FX = 110.9
FY = 110.9
CX = 64.0
CY = 64.0
NEAR = 0.01
FAR = 100.0
MAX_RADIUS = 32.0


def _b16(x):
    return x.astype(jnp.bfloat16).astype(jnp.float32)


def _project_kernel(view_ref, px_ref, py_ref, pz_ref, sx_ref, sy_ref, sz_ref,
                    qw_ref, qx_ref, qy_ref, qz_ref, op_ref, out_ref):
    v = view_ref
    vb = [[_b16(v[i, j]) for j in range(4)] for i in range(4)]
    px = _b16(px_ref[...])
    py = _b16(py_ref[...])
    pz = _b16(pz_ref[...])
    pcx = vb[0][0] * px + vb[0][1] * py + vb[0][2] * pz + vb[0][3]
    pcy = vb[1][0] * px + vb[1][1] * py + vb[1][2] * pz + vb[1][3]
    pcz = vb[2][0] * px + vb[2][1] * py + vb[2][2] * pz + vb[2][3]
    depth = -pcz

    qw = qw_ref[...]
    qx = qx_ref[...]
    qy = qy_ref[...]
    qz = qz_ref[...]
    qn = jnp.sqrt(qw * qw + qx * qx + qy * qy + qz * qz) + 1e-12
    w = qw / qn
    x = qx / qn
    y = qy / qn
    z = qz / qn
    r = [[1 - 2 * y * y - 2 * z * z, 2 * x * y - 2 * w * z, 2 * x * z + 2 * w * y],
         [2 * x * y + 2 * w * z, 1 - 2 * x * x - 2 * z * z, 2 * y * z - 2 * w * x],
         [2 * x * z - 2 * w * y, 2 * y * z + 2 * w * x, 1 - 2 * x * x - 2 * y * y]]
    s = [_b16(sx_ref[...]), _b16(sy_ref[...]), _b16(sz_ref[...])]
    rc = [[vb[i][0] * _b16(r[0][j]) + vb[i][1] * _b16(r[1][j])
           + vb[i][2] * _b16(r[2][j]) for j in range(3)] for i in range(3)]
    rs = [[_b16(rc[i][j]) * s[j] for j in range(3)] for i in range(3)]
    rsb = [[_b16(rs[i][j]) for j in range(3)] for i in range(3)]
    cov = [[rsb[i][0] * rsb[j][0] + rsb[i][1] * rsb[j][1] + rsb[i][2] * rsb[j][2]
            for j in range(3)] for i in range(3)]

    zsafe = jnp.maximum(jnp.abs(pcz), 0.01) * jnp.sign(pcz + 1e-8)
    z2 = zsafe * zsafe
    j00 = FX / -zsafe
    j02 = FX * pcx / z2
    j11 = FY / zsafe
    j12 = FY * pcy / z2
    j00b = _b16(j00)
    j02b = _b16(j02)
    j11b = _b16(j11)
    j12b = _b16(j12)
    covb = [[_b16(cov[i][j]) for j in range(3)] for i in range(3)]
    t00 = j00b * covb[0][0] + j02b * covb[2][0]
    t01 = j00b * covb[0][1] + j02b * covb[2][1]
    t02 = j00b * covb[0][2] + j02b * covb[2][2]
    t10 = j11b * covb[1][0] + j12b * covb[2][0]
    t11 = j11b * covb[1][1] + j12b * covb[2][1]
    t12 = j11b * covb[1][2] + j12b * covb[2][2]
    a = _b16(t00) * j00b + _b16(t02) * j02b
    b = _b16(t01) * j11b + _b16(t02) * j12b
    c = _b16(t10) * j00b + _b16(t12) * j02b
    d = _b16(t11) * j11b + _b16(t12) * j12b

    u = FX * pcx / -zsafe + CX
    vv = FY * -pcy / -zsafe + CY
    trace = a + d
    det = jnp.maximum(a * d - b * c, 1e-6)
    disc = jnp.maximum(trace * trace - 4.0 * det, 0.0)
    max_eig = (trace + jnp.sqrt(disc)) / 2.0
    radii = jnp.minimum(3.0 * jnp.sqrt(jnp.maximum(max_eig, 1e-6)), MAX_RADIUS)

    vis = ((depth > NEAR) & (depth < FAR)
           & (u + radii > 0) & (u - radii < W_IMG)
           & (vv + radii > 0) & (vv - radii < H_IMG))

    ar = a + 0.3
    dr = d + 0.3
    br = b
    det_r = jnp.maximum(ar * dr - br * br, 1e-6)
    inv_a = dr / det_r
    inv_d = ar / det_r
    inv_b = -br / det_r
    oe = op_ref[...] * vis.astype(jnp.float32)

    out_ref[0] = depth
    out_ref[1] = u
    out_ref[2] = vv
    out_ref[3] = inv_a
    out_ref[4] = inv_b
    out_ref[5] = inv_d
    out_ref[6] = oe
    out_ref[7] = radii


def _rank_kernel(dcol_ref, drow_ref, icol_ref, irow_ref, out_ref):
    dcol = dcol_ref[...]
    drow = drow_ref[...]
    icol = icol_ref[...]
    irow = irow_ref[...]
    lt = jnp.where((dcol < drow) | ((dcol == drow) & (icol < irow)), 1.0, 0.0)
    out_ref[...] = jnp.sum(lt, axis=0, keepdims=True)


_SC_TILES = 32
_ROWS_PER_TILE = N_G // _SC_TILES


_MCOLS = 128


def _sc_sort_scatter(m_hbm, rank_hbm, out_hbm, idx_v, rows_v, sem):
    wid = lax.axis_index("s") * 2 + lax.axis_index("c")
    base = wid * _ROWS_PER_TILE
    pltpu.sync_copy(rank_hbm.at[pl.ds(base, _ROWS_PER_TILE)], idx_v)
    pltpu.sync_copy(m_hbm.at[pl.ds(base, _ROWS_PER_TILE)], rows_v)
    pltpu.async_copy(rows_v, out_hbm.at[idx_v], sem).wait()


def _sorted_params(depth, m):
    f32 = jnp.float32
    dcol = depth.reshape(N_G, 1)
    drow = depth.reshape(1, N_G)
    idx = jnp.arange(N_G, dtype=f32)
    icol = idx.reshape(N_G, 1)
    irow = idx.reshape(1, N_G)
    rank = pl.pallas_call(
        _rank_kernel,
        out_shape=jax.ShapeDtypeStruct((1, N_G), f32),
    )(dcol, drow, icol, irow)
    rank_i = rank.reshape(N_G).astype(jnp.int32)

    mesh = plsc.VectorSubcoreMesh(core_axis_name="c", subcore_axis_name="s")
    scatter = functools.partial(
        pl.kernel, mesh=mesh,
        out_type=jax.ShapeDtypeStruct((N_G, _MCOLS), f32),
        scratch_types=[
            pltpu.VMEM((_ROWS_PER_TILE,), jnp.int32),
            pltpu.VMEM((_ROWS_PER_TILE, _MCOLS), f32),
            pltpu.SemaphoreType.DMA,
        ],
    )(_sc_sort_scatter)
    return scatter(m, rank_i)[:, :16]


_WROWS = 80


def _raster_kernel(ms_ref, out_ref, t_ref):
    t_ref[...] = jnp.ones((H_IMG, W_IMG), jnp.float32)
    out_ref[...] = jnp.zeros((3, H_IMG, W_IMG), jnp.float32)

    xs = jax.lax.broadcasted_iota(
        jnp.int32, (_WROWS, W_IMG), 1).astype(jnp.float32) + 0.5

    def alpha_for(g):
        u = ms_ref[g, 0]
        v = ms_ref[g, 1]
        inv_a = ms_ref[g, 2]
        inv_b = ms_ref[g, 3]
        inv_d = ms_ref[g, 4]
        oe = ms_ref[g, 5]
        rad = ms_ref[g, 6]
        r2 = rad * rad
        lo = jnp.maximum(jnp.floor(v - rad).astype(jnp.int32) - 1, 0)
        lo = (lo // 8) * 8
        start = pl.multiple_of(jnp.clip(lo, 0, H_IMG - _WROWS), 8)
        ys = (jax.lax.broadcasted_iota(jnp.int32, (_WROWS, W_IMG), 0)
              + start).astype(jnp.float32) + 0.5
        dx = xs - u
        dy = ys - v
        dx2 = dx * dx
        dy2 = dy * dy
        power = -0.5 * (inv_a * dx2 + inv_d * dy2) - inv_b * (dx * dy)
        power = jnp.minimum(power, 0.0)
        gauss = jnp.exp(power)
        alpha = jnp.where(dx2 + dy2 <= r2, oe * gauss, 0.0)
        return start, jnp.minimum(alpha, 0.99)

    def apply(g, start, alpha):
        c0b = _b16(ms_ref[g, 7])
        c1b = _b16(ms_ref[g, 8])
        c2b = _b16(ms_ref[g, 9])
        tcur = t_ref[pl.ds(start, _WROWS), :]
        wgt = _b16(tcur * alpha)
        out_ref[0, pl.ds(start, _WROWS), :] += wgt * c0b
        out_ref[1, pl.ds(start, _WROWS), :] += wgt * c1b
        out_ref[2, pl.ds(start, _WROWS), :] += wgt * c2b
        t_ref[pl.ds(start, _WROWS), :] = tcur * (1.0 - alpha)

    def body(gp, carry):
        g0 = 4 * gp
        s0, a0 = alpha_for(g0)
        s1, a1 = alpha_for(g0 + 1)
        apply(g0, s0, a0)
        s2, a2 = alpha_for(g0 + 2)
        apply(g0 + 1, s1, a1)
        s3, a3 = alpha_for(g0 + 3)
        apply(g0 + 2, s2, a2)
        apply(g0 + 3, s3, a3)
        return carry

    jax.lax.fori_loop(0, N_G // 4, body, 0)


def kernel(positions, scales, rotations, colors, opacities, view_matrix):
    f32 = jnp.float32
    px = positions[:, 0].reshape(8, 128)
    py = positions[:, 1].reshape(8, 128)
    pz = positions[:, 2].reshape(8, 128)
    sx = scales[:, 0].reshape(8, 128)
    sy = scales[:, 1].reshape(8, 128)
    sz = scales[:, 2].reshape(8, 128)
    qw = rotations[:, 0].reshape(8, 128)
    qx = rotations[:, 1].reshape(8, 128)
    qy = rotations[:, 2].reshape(8, 128)
    qz = rotations[:, 3].reshape(8, 128)
    op = opacities.reshape(8, 128)

    proj = pl.pallas_call(
        _project_kernel,
        out_shape=jax.ShapeDtypeStruct((8, 8, 128), f32),
        in_specs=[pl.BlockSpec(memory_space=pltpu.SMEM)]
                 + [pl.BlockSpec(memory_space=pltpu.VMEM)] * 11,
    )(view_matrix, px, py, pz, sx, sy, sz, qw, qx, qy, qz, op)

    flat = proj.reshape(8, N_G)
    depth = flat[0]
    m = jnp.concatenate(
        [flat[1:8].T, colors, jnp.zeros((N_G, _MCOLS - 10), f32)], axis=1)
    ms = _sorted_params(depth, m)

    img = pl.pallas_call(
        _raster_kernel,
        out_shape=jax.ShapeDtypeStruct((3, H_IMG, W_IMG), f32),
        in_specs=[pl.BlockSpec(memory_space=pltpu.SMEM)],
        scratch_shapes=[pltpu.VMEM((H_IMG, W_IMG), f32)],
    )(ms)
    return jnp.transpose(img, (1, 2, 0))
